# baseline (speedup 1.0000x reference)
import jax
import jax.numpy as jnp
from jax import lax
from jax.experimental import pallas as pl
from jax.experimental.pallas import tpu as pltpu
from jax.experimental.pallas import tpu_sc as plsc

N = 10000
E = 160000
H = 256
OUT = 128
IB = 128
ROWS = 1280
EPAD = ROWS * IB - E
NA = N + 16
NPT = 624
NBUF = 2
NBUF2 = 4
CH = 40


def _mm1a_body(xu_ref, xi_ref, wf_ref, wrb_ref,
               f_lo, f_hi, rb_lo, rb_hi):
    fh = H // 2
    mf = jnp.dot(xu_ref[...], wf_ref[...], preferred_element_type=jnp.float32)
    mrb = jnp.dot(xi_ref[...], wrb_ref[...],
                  preferred_element_type=jnp.float32)
    f_lo[...] = mf[:, :fh]
    f_hi[...] = mf[:, fh:]
    rb_lo[...] = mrb[:, :fh]
    rb_hi[...] = mrb[:, fh:]


def _make_mm1a():
    bm = 1000
    half = jax.ShapeDtypeStruct((N, H // 2), jnp.float32)
    return pl.pallas_call(
        _mm1a_body,
        grid=(N // bm,),
        in_specs=[
            pl.BlockSpec((bm, H), lambda i: (i, 0)),
            pl.BlockSpec((bm, H), lambda i: (i, 0)),
            pl.BlockSpec((H, H), lambda i: (0, 0)),
            pl.BlockSpec((H, H), lambda i: (0, 0)),
        ],
        out_specs=[pl.BlockSpec((bm, H // 2), lambda i: (i, 0))] * 4,
        out_shape=[half] * 4,
        compiler_params=pltpu.CompilerParams(
            dimension_semantics=("parallel",)),
    )


def _mm1b_body(xu_ref, wr_ref, r_lo, r_hi):
    fh = H // 2
    mr = jnp.dot(xu_ref[...], wr_ref[...], preferred_element_type=jnp.float32)
    r_lo[...] = mr[:, :fh]
    r_hi[...] = mr[:, fh:]


def _make_mm1b():
    bm = 1000
    half = jax.ShapeDtypeStruct((N, H // 2), jnp.float32)
    return pl.pallas_call(
        _mm1b_body,
        grid=(N // bm,),
        in_specs=[
            pl.BlockSpec((bm, H), lambda i: (i, 0)),
            pl.BlockSpec((H, H), lambda i: (0, 0)),
        ],
        out_specs=[pl.BlockSpec((bm, H // 2), lambda i: (i, 0))] * 2,
        out_shape=[half] * 2,
        compiler_params=pltpu.CompilerParams(
            dimension_semantics=("parallel",)),
    )


def _mm2a_body(xu_ref, wf_ref, wr_ref, f_lo, f_hi, r_lo, r_hi):
    fh = OUT // 2
    xu = jnp.maximum(xu_ref[...], 0.0)
    mf = jnp.dot(xu, wf_ref[...], preferred_element_type=jnp.float32)
    mr = jnp.dot(xu, wr_ref[...], preferred_element_type=jnp.float32)
    f_lo[...] = mf[:, :fh]
    f_hi[...] = mf[:, fh:]
    r_lo[...] = mr[:, :fh]
    r_hi[...] = mr[:, fh:]


def _make_mm2a():
    bm = 1000
    quarter = jax.ShapeDtypeStruct((N, OUT // 2), jnp.float32)
    return pl.pallas_call(
        _mm2a_body,
        grid=(N // bm,),
        in_specs=[
            pl.BlockSpec((bm, H), lambda i: (i, 0)),
            pl.BlockSpec((H, OUT), lambda i: (0, 0)),
            pl.BlockSpec((H, OUT), lambda i: (0, 0)),
        ],
        out_specs=[pl.BlockSpec((bm, OUT // 2), lambda i: (i, 0))] * 4,
        out_shape=[quarter] * 4,
        compiler_params=pltpu.CompilerParams(
            dimension_semantics=("parallel",)),
    )


def _mm2b_body(xi_ref, wrb_ref, rb_lo, rb_hi):
    fh = OUT // 2
    xi = jnp.maximum(xi_ref[...], 0.0)
    mrb = jnp.dot(xi, wrb_ref[...], preferred_element_type=jnp.float32)
    rb_lo[...] = mrb[:, :fh]
    rb_hi[...] = mrb[:, fh:]


def _make_mm2b():
    bm = 1000
    quarter = jax.ShapeDtypeStruct((N, OUT // 2), jnp.float32)
    return pl.pallas_call(
        _mm2b_body,
        grid=(N // bm,),
        in_specs=[
            pl.BlockSpec((bm, H), lambda i: (i, 0)),
            pl.BlockSpec((H, OUT), lambda i: (0, 0)),
        ],
        out_specs=[pl.BlockSpec((bm, OUT // 2), lambda i: (i, 0))] * 2,
        out_shape=[quarter] * 2,
        compiler_params=pltpu.CompilerParams(
            dimension_semantics=("parallel",)),
    )


def _merge_relu_body(ul_ref, uh_ref, il_ref, ih_ref, ou_ref, oi_ref):
    ou_ref[...] = jnp.maximum(
        jnp.concatenate([ul_ref[...], uh_ref[...]], axis=1), 0.0)
    oi_ref[...] = jnp.maximum(
        jnp.concatenate([il_ref[...], ih_ref[...]], axis=1), 0.0)


def _make_merge_relu():
    bm = 1000
    shp = jax.ShapeDtypeStruct((N, OUT), jnp.float32)
    qspec = pl.BlockSpec((bm, OUT // 2), lambda i: (i, 0))
    return pl.pallas_call(
        _merge_relu_body,
        grid=(N // bm,),
        in_specs=[qspec] * 4,
        out_specs=[pl.BlockSpec((bm, OUT), lambda i: (i, 0))] * 2,
        out_shape=[shp, shp],
        compiler_params=pltpu.CompilerParams(
            dimension_semantics=("parallel",)),
    )


_MESH = plsc.VectorSubcoreMesh(core_axis_name="c", subcore_axis_name="s",
                               num_cores=2)


def _clear_accs(zbuf, accs, s, w):
    zero = jnp.zeros((16,), jnp.float32)

    def zrow(r, carry):
        for j in range(w // 16):
            zbuf[r, pl.ds(16 * j, 16)] = zero
        return carry

    lax.fori_loop(0, IB, zrow, 0)
    for acc in accs:
        for k in range(4):
            pltpu.sync_copy(zbuf.at[pl.ds(0, 128), :],
                            acc.at[pl.ds(s * NPT + k * 128, 128), :])
        pltpu.sync_copy(zbuf.at[pl.ds(0, 112), :],
                        acc.at[pl.ds(s * NPT + 512, 112), :])

        @pl.when(s == 15)
        def _():
            pltpu.sync_copy(zbuf.at[pl.ds(0, 16), :],
                            acc.at[pl.ds(16 * NPT, 16), :])


def _accumulate(h_ref, ei_ref, sidx, didx, rows, gsems, ssems, acc,
                base, nrows):
    nb = len(rows)
    ngrp = CH // nb

    def chunk(ci, carry):
        cbase = base + ci * CH
        pltpu.sync_copy(ei_ref.at[0, pl.ds(cbase, CH), :], sidx)
        pltpu.sync_copy(ei_ref.at[1, pl.ds(cbase, CH), :], didx)

        for b in range(nb):
            pltpu.async_copy(h_ref.at[sidx.at[b]], rows[b], gsems[b])

        def group(g, c2):
            t0 = g * nb
            for b in range(nb):
                pltpu.make_async_copy(h_ref.at[sidx.at[t0 + b]], rows[b],
                                      gsems[b]).wait()
                pltpu.async_copy(rows[b], acc.at[didx.at[t0 + b]], ssems[b],
                                 add=True)
            for b in range(nb):
                pltpu.make_async_copy(rows[b], acc.at[didx.at[t0 + b]],
                                      ssems[b]).wait()

                @pl.when(g + 1 < ngrp)
                def _():
                    pltpu.async_copy(h_ref.at[sidx.at[t0 + nb + b]],
                                     rows[b], gsems[b])
            return c2

        lax.fori_loop(0, ngrp, group, 0)
        return carry

    lax.fori_loop(0, nrows // CH, chunk, 0)


def _make_spmm_featsplit(n_rel):
    fh = H // 2
    rpt = ROWS // 16

    def body(*refs):
        h_refs = refs[:2 * n_rel]
        ei_refs = refs[2 * n_rel:3 * n_rel]
        out_ref = refs[3 * n_rel]
        sidx = refs[3 * n_rel + 1]
        didx = refs[3 * n_rel + 2]
        rows = refs[3 * n_rel + 3:3 * n_rel + 3 + NBUF]
        acc = refs[3 * n_rel + 3 + NBUF]
        gsems = refs[3 * n_rel + 4 + NBUF:3 * n_rel + 4 + 2 * NBUF]
        ssems = refs[3 * n_rel + 4 + 2 * NBUF:]

        c = lax.axis_index("c")
        s = lax.axis_index("s")

        _clear_accs(rows[0], [acc], s, 128)
        plsc.subcore_barrier()

        @pl.when(c == 0)
        def _():
            for r in range(n_rel):
                _accumulate(h_refs[2 * r], ei_refs[r], sidx, didx, rows,
                            gsems, ssems, acc, s * rpt, rpt)

        @pl.when(c == 1)
        def _():
            for r in range(n_rel):
                _accumulate(h_refs[2 * r + 1], ei_refs[r], sidx, didx, rows,
                            gsems, ssems, acc, s * rpt, rpt)

        plsc.subcore_barrier()
        pltpu.sync_copy(acc.at[pl.ds(s * NPT, NPT), :],
                        out_ref.at[pl.ds(s * NPT, NPT), pl.ds(c * fh, fh)])

        @pl.when(s == 15)
        def _():
            pltpu.sync_copy(acc.at[pl.ds(16 * NPT, 16), :],
                            out_ref.at[pl.ds(16 * NPT, 16), pl.ds(c * fh, fh)])

    return pl.kernel(
        body,
        out_type=jax.ShapeDtypeStruct((N, H), jnp.float32),
        mesh=_MESH,
        compiler_params=pltpu.CompilerParams(use_tc_tiling_on_sc=False),
        scratch_types=(
            [pltpu.VMEM((CH, IB), jnp.int32)] * 2 +
            [pltpu.VMEM((IB, fh), jnp.float32)] * NBUF +
            [pltpu.VMEM_SHARED((NA, fh), jnp.float32)] +
            [pltpu.SemaphoreType.DMA] * (2 * NBUF)
        ),
    )


def _make_spmm_l2():
    fh = OUT // 2
    rpt = ROWS // 16

    def body(gf_lo, gf_hi, gr_lo, gr_hi, grb_lo, grb_hi,
             ei_f, ei_r, ei_rb, ou_lo, ou_hi, oi_lo, oi_hi,
             sidx, didx, *rest):
        rows = rest[:NBUF2]
        acc_u = rest[NBUF2]
        acc_i = rest[NBUF2 + 1]
        gsems = rest[NBUF2 + 2:NBUF2 + 2 + NBUF2]
        ssems = rest[NBUF2 + 2 + NBUF2:]

        c = lax.axis_index("c")
        s = lax.axis_index("s")

        _clear_accs(rows[0], [acc_u, acc_i], s, fh)
        plsc.subcore_barrier()

        def run(gf, gr, grb):
            _accumulate(gf, ei_f, sidx, didx, rows, gsems, ssems, acc_u,
                        s * rpt, rpt)
            _accumulate(grb, ei_rb, sidx, didx, rows, gsems, ssems, acc_u,
                        s * rpt, rpt)
            _accumulate(gr, ei_r, sidx, didx, rows, gsems, ssems, acc_i,
                        s * rpt, rpt)

        @pl.when(c == 0)
        def _():
            run(gf_lo, gr_lo, grb_lo)

        @pl.when(c == 1)
        def _():
            run(gf_hi, gr_hi, grb_hi)

        plsc.subcore_barrier()

        def wout(acc, o_lo, o_hi):
            @pl.when(c == 0)
            def _():
                pltpu.sync_copy(acc.at[pl.ds(s * NPT, NPT), :],
                                o_lo.at[pl.ds(s * NPT, NPT), :])

                @pl.when(s == 15)
                def _():
                    pltpu.sync_copy(acc.at[pl.ds(16 * NPT, 16), :],
                                    o_lo.at[pl.ds(16 * NPT, 16), :])

            @pl.when(c == 1)
            def _():
                pltpu.sync_copy(acc.at[pl.ds(s * NPT, NPT), :],
                                o_hi.at[pl.ds(s * NPT, NPT), :])

                @pl.when(s == 15)
                def _():
                    pltpu.sync_copy(acc.at[pl.ds(16 * NPT, 16), :],
                                    o_hi.at[pl.ds(16 * NPT, 16), :])

        wout(acc_u, ou_lo, ou_hi)
        wout(acc_i, oi_lo, oi_hi)

    quarter = jax.ShapeDtypeStruct((N, fh), jnp.float32)
    return pl.kernel(
        body,
        out_type=[quarter] * 4,
        mesh=_MESH,
        compiler_params=pltpu.CompilerParams(use_tc_tiling_on_sc=False),
        scratch_types=(
            [pltpu.VMEM((CH, IB), jnp.int32)] * 2 +
            [pltpu.VMEM((IB, fh), jnp.float32)] * NBUF2 +
            [pltpu.VMEM_SHARED((NA, fh), jnp.float32)] * 2 +
            [pltpu.SemaphoreType.DMA] * (2 * NBUF2)
        ),
    )


def _pad_edges(ei):
    r = jnp.arange(EPAD, dtype=jnp.int32)
    pad = jnp.stack([r % N, N + (r % (NA - N))])
    return jnp.concatenate([ei.astype(jnp.int32), pad], axis=1).reshape(
        2, ROWS, IB)


def kernel(x_user, x_item, W1_follows, W1_rates, W1_ratedby,
           W2_follows, W2_rates, W2_ratedby,
           ei_follows, ei_rates, ei_ratedby):
    ei_f = _pad_edges(ei_follows)
    ei_r = _pad_edges(ei_rates)
    ei_rb = _pad_edges(ei_ratedby)

    mm1a = _make_mm1a()
    mm1b = _make_mm1b()
    mm2a = _make_mm2a()
    mm2b = _make_mm2b()
    spmm2_h = _make_spmm_featsplit(2)
    spmm1_h = _make_spmm_featsplit(1)
    spmm_l2 = _make_spmm_l2()
    merge_relu = _make_merge_relu()

    hf_lo, hf_hi, hrb_lo, hrb_hi = mm1a(x_user, x_item, W1_follows,
                                        W1_ratedby)
    hr_lo, hr_hi = mm1b(x_user, W1_rates)
    hu1 = spmm2_h(hf_lo, hf_hi, hrb_lo, hrb_hi, ei_f, ei_rb)
    dep1 = (hu1[0, 0] * 0.0).astype(jnp.int32)
    hi1 = spmm1_h(hr_lo, hr_hi, ei_r + dep1)

    gf_lo, gf_hi, gr_lo, gr_hi = mm2a(hu1, W2_follows, W2_rates)
    grb_lo, grb_hi = mm2b(hi1, W2_ratedby)
    hu2_lo, hu2_hi, hi2_lo, hi2_hi = spmm_l2(
        gf_lo, gf_hi, gr_lo, gr_hi, grb_lo, grb_hi, ei_f, ei_r, ei_rb)

    return merge_relu(hu2_lo, hu2_hi, hi2_lo, hi2_hi)


# speedup vs baseline: 1.0472x; 1.0472x over previous
import jax
import jax.numpy as jnp
from jax import lax
from jax.experimental import pallas as pl
from jax.experimental.pallas import tpu as pltpu
from jax.experimental.pallas import tpu_sc as plsc

N = 10000
E = 160000
H = 256
OUT = 128
IB = 128
ROWS = 1280
EPAD = ROWS * IB - E
NA = N + 16
NPT = 624
NBUF = 2
NBUF2 = 4
CH = 40


def _mm1a_body(xu_ref, xi_ref, wf_ref, wrb_ref, *outs):
    fq = H // 4
    mf = jnp.dot(xu_ref[...], wf_ref[...], preferred_element_type=jnp.float32)
    mrb = jnp.dot(xi_ref[...], wrb_ref[...],
                  preferred_element_type=jnp.float32)
    for q in range(4):
        outs[q][...] = mf[:, q * fq:(q + 1) * fq]
        outs[4 + q][...] = mrb[:, q * fq:(q + 1) * fq]


def _make_mm1a():
    bm = 1000
    quarter = jax.ShapeDtypeStruct((N, H // 4), jnp.float32)
    return pl.pallas_call(
        _mm1a_body,
        grid=(N // bm,),
        in_specs=[
            pl.BlockSpec((bm, H), lambda i: (i, 0)),
            pl.BlockSpec((bm, H), lambda i: (i, 0)),
            pl.BlockSpec((H, H), lambda i: (0, 0)),
            pl.BlockSpec((H, H), lambda i: (0, 0)),
        ],
        out_specs=[pl.BlockSpec((bm, H // 4), lambda i: (i, 0))] * 8,
        out_shape=[quarter] * 8,
        compiler_params=pltpu.CompilerParams(
            dimension_semantics=("parallel",)),
    )


def _mm1b_body(xu_ref, wr_ref, *outs):
    fq = H // 4
    mr = jnp.dot(xu_ref[...], wr_ref[...], preferred_element_type=jnp.float32)
    for q in range(4):
        outs[q][...] = mr[:, q * fq:(q + 1) * fq]


def _make_mm1b():
    bm = 1000
    quarter = jax.ShapeDtypeStruct((N, H // 4), jnp.float32)
    return pl.pallas_call(
        _mm1b_body,
        grid=(N // bm,),
        in_specs=[
            pl.BlockSpec((bm, H), lambda i: (i, 0)),
            pl.BlockSpec((H, H), lambda i: (0, 0)),
        ],
        out_specs=[pl.BlockSpec((bm, H // 4), lambda i: (i, 0))] * 4,
        out_shape=[quarter] * 4,
        compiler_params=pltpu.CompilerParams(
            dimension_semantics=("parallel",)),
    )


def _mm2a_body(q0, q1, q2, q3, wf_ref, wr_ref, f_lo, f_hi, r_lo, r_hi):
    fh = OUT // 2
    xu = jnp.maximum(jnp.concatenate(
        [q0[...], q1[...], q2[...], q3[...]], axis=1), 0.0)
    mf = jnp.dot(xu, wf_ref[...], preferred_element_type=jnp.float32)
    mr = jnp.dot(xu, wr_ref[...], preferred_element_type=jnp.float32)
    f_lo[...] = mf[:, :fh]
    f_hi[...] = mf[:, fh:]
    r_lo[...] = mr[:, :fh]
    r_hi[...] = mr[:, fh:]


def _make_mm2a():
    bm = 1000
    quarter = jax.ShapeDtypeStruct((N, OUT // 2), jnp.float32)
    return pl.pallas_call(
        _mm2a_body,
        grid=(N // bm,),
        in_specs=[pl.BlockSpec((bm, H // 4), lambda i: (i, 0))] * 4 + [
            pl.BlockSpec((H, OUT), lambda i: (0, 0)),
            pl.BlockSpec((H, OUT), lambda i: (0, 0)),
        ],
        out_specs=[pl.BlockSpec((bm, OUT // 2), lambda i: (i, 0))] * 4,
        out_shape=[quarter] * 4,
        compiler_params=pltpu.CompilerParams(
            dimension_semantics=("parallel",)),
    )


def _mm2b_body(q0, q1, q2, q3, wrb_ref, rb_lo, rb_hi):
    fh = OUT // 2
    xi = jnp.maximum(jnp.concatenate(
        [q0[...], q1[...], q2[...], q3[...]], axis=1), 0.0)
    mrb = jnp.dot(xi, wrb_ref[...], preferred_element_type=jnp.float32)
    rb_lo[...] = mrb[:, :fh]
    rb_hi[...] = mrb[:, fh:]


def _make_mm2b():
    bm = 1000
    quarter = jax.ShapeDtypeStruct((N, OUT // 2), jnp.float32)
    return pl.pallas_call(
        _mm2b_body,
        grid=(N // bm,),
        in_specs=[pl.BlockSpec((bm, H // 4), lambda i: (i, 0))] * 4 + [
            pl.BlockSpec((H, OUT), lambda i: (0, 0)),
        ],
        out_specs=[pl.BlockSpec((bm, OUT // 2), lambda i: (i, 0))] * 2,
        out_shape=[quarter] * 2,
        compiler_params=pltpu.CompilerParams(
            dimension_semantics=("parallel",)),
    )


def _merge_relu_body(ul_ref, uh_ref, il_ref, ih_ref, ou_ref, oi_ref):
    ou_ref[...] = jnp.maximum(
        jnp.concatenate([ul_ref[...], uh_ref[...]], axis=1), 0.0)
    oi_ref[...] = jnp.maximum(
        jnp.concatenate([il_ref[...], ih_ref[...]], axis=1), 0.0)


def _make_merge_relu():
    bm = 1000
    shp = jax.ShapeDtypeStruct((N, OUT), jnp.float32)
    qspec = pl.BlockSpec((bm, OUT // 2), lambda i: (i, 0))
    return pl.pallas_call(
        _merge_relu_body,
        grid=(N // bm,),
        in_specs=[qspec] * 4,
        out_specs=[pl.BlockSpec((bm, OUT), lambda i: (i, 0))] * 2,
        out_shape=[shp, shp],
        compiler_params=pltpu.CompilerParams(
            dimension_semantics=("parallel",)),
    )


_MESH = plsc.VectorSubcoreMesh(core_axis_name="c", subcore_axis_name="s",
                               num_cores=2)


def _clear_accs(zbuf, accs, s, w):
    zero = jnp.zeros((16,), jnp.float32)

    def zrow(r, carry):
        for j in range(w // 16):
            zbuf[r, pl.ds(16 * j, 16)] = zero
        return carry

    lax.fori_loop(0, IB, zrow, 0)
    for acc in accs:
        for k in range(4):
            pltpu.sync_copy(zbuf.at[pl.ds(0, 128), :],
                            acc.at[pl.ds(s * NPT + k * 128, 128), :])
        pltpu.sync_copy(zbuf.at[pl.ds(0, 112), :],
                        acc.at[pl.ds(s * NPT + 512, 112), :])

        @pl.when(s == 15)
        def _():
            pltpu.sync_copy(zbuf.at[pl.ds(0, 16), :],
                            acc.at[pl.ds(16 * NPT, 16), :])


def _accumulate(h_ref, ei_ref, sidx, didx, rows, gsems, ssems, acc,
                base, nrows):
    nb = len(rows)
    ngrp = CH // nb

    def chunk(ci, carry):
        cbase = base + ci * CH
        pltpu.sync_copy(ei_ref.at[0, pl.ds(cbase, CH), :], sidx)
        pltpu.sync_copy(ei_ref.at[1, pl.ds(cbase, CH), :], didx)

        for b in range(nb):
            pltpu.async_copy(h_ref.at[sidx.at[b]], rows[b], gsems[b])

        def group(g, c2):
            t0 = g * nb
            for b in range(nb):
                pltpu.make_async_copy(h_ref.at[sidx.at[t0 + b]], rows[b],
                                      gsems[b]).wait()
                pltpu.async_copy(rows[b], acc.at[didx.at[t0 + b]], ssems[b],
                                 add=True)
            for b in range(nb):
                pltpu.make_async_copy(rows[b], acc.at[didx.at[t0 + b]],
                                      ssems[b]).wait()

                @pl.when(g + 1 < ngrp)
                def _():
                    pltpu.async_copy(h_ref.at[sidx.at[t0 + nb + b]],
                                     rows[b], gsems[b])
            return c2

        lax.fori_loop(0, ngrp, group, 0)
        return carry

    lax.fori_loop(0, nrows // CH, chunk, 0)


def _make_spmm_l1(n_rel):
    fq = H // 4
    rpt = ROWS // 16

    def body(*refs):
        nq = 4 * n_rel
        h_refs = refs[:nq]
        ei_refs = refs[nq:nq + n_rel]
        out_refs = refs[nq + n_rel:nq + n_rel + 4]
        sidx = refs[nq + n_rel + 4]
        didx = refs[nq + n_rel + 5]
        rest = refs[nq + n_rel + 6:]
        rows = rest[:NBUF2]
        acc_a = rest[NBUF2]
        acc_b = rest[NBUF2 + 1]
        gsems = rest[NBUF2 + 2:NBUF2 + 2 + NBUF2]
        ssems = rest[NBUF2 + 2 + NBUF2:]

        c = lax.axis_index("c")
        s = lax.axis_index("s")

        _clear_accs(rows[0], [acc_a, acc_b], s, fq)
        plsc.subcore_barrier()

        def run(qoff):
            for r in range(n_rel):
                _accumulate(h_refs[4 * r + qoff], ei_refs[r], sidx, didx,
                            rows, gsems, ssems, acc_a, s * rpt, rpt)
                _accumulate(h_refs[4 * r + qoff + 1], ei_refs[r], sidx, didx,
                            rows, gsems, ssems, acc_b, s * rpt, rpt)

        @pl.when(c == 0)
        def _():
            run(0)

        @pl.when(c == 1)
        def _():
            run(2)

        plsc.subcore_barrier()

        def wout(acc, o0, o1):
            @pl.when(c == 0)
            def _():
                pltpu.sync_copy(acc.at[pl.ds(s * NPT, NPT), :],
                                o0.at[pl.ds(s * NPT, NPT), :])

                @pl.when(s == 15)
                def _():
                    pltpu.sync_copy(acc.at[pl.ds(16 * NPT, 16), :],
                                    o0.at[pl.ds(16 * NPT, 16), :])

            @pl.when(c == 1)
            def _():
                pltpu.sync_copy(acc.at[pl.ds(s * NPT, NPT), :],
                                o1.at[pl.ds(s * NPT, NPT), :])

                @pl.when(s == 15)
                def _():
                    pltpu.sync_copy(acc.at[pl.ds(16 * NPT, 16), :],
                                    o1.at[pl.ds(16 * NPT, 16), :])

        wout(acc_a, out_refs[0], out_refs[2])
        wout(acc_b, out_refs[1], out_refs[3])

    quarter = jax.ShapeDtypeStruct((N, fq), jnp.float32)
    return pl.kernel(
        body,
        out_type=[quarter] * 4,
        mesh=_MESH,
        compiler_params=pltpu.CompilerParams(use_tc_tiling_on_sc=False),
        scratch_types=(
            [pltpu.VMEM((CH, IB), jnp.int32)] * 2 +
            [pltpu.VMEM((IB, fq), jnp.float32)] * NBUF2 +
            [pltpu.VMEM_SHARED((NA, fq), jnp.float32)] * 2 +
            [pltpu.SemaphoreType.DMA] * (2 * NBUF2)
        ),
    )


def _make_spmm_l2():
    fh = OUT // 2
    rpt = ROWS // 16

    def body(gf_lo, gf_hi, gr_lo, gr_hi, grb_lo, grb_hi,
             ei_f, ei_r, ei_rb, ou_lo, ou_hi, oi_lo, oi_hi,
             sidx, didx, *rest):
        rows = rest[:NBUF2]
        acc_u = rest[NBUF2]
        acc_i = rest[NBUF2 + 1]
        gsems = rest[NBUF2 + 2:NBUF2 + 2 + NBUF2]
        ssems = rest[NBUF2 + 2 + NBUF2:]

        c = lax.axis_index("c")
        s = lax.axis_index("s")

        _clear_accs(rows[0], [acc_u, acc_i], s, fh)
        plsc.subcore_barrier()

        def run(gf, gr, grb):
            _accumulate(gf, ei_f, sidx, didx, rows, gsems, ssems, acc_u,
                        s * rpt, rpt)
            _accumulate(grb, ei_rb, sidx, didx, rows, gsems, ssems, acc_u,
                        s * rpt, rpt)
            _accumulate(gr, ei_r, sidx, didx, rows, gsems, ssems, acc_i,
                        s * rpt, rpt)

        @pl.when(c == 0)
        def _():
            run(gf_lo, gr_lo, grb_lo)

        @pl.when(c == 1)
        def _():
            run(gf_hi, gr_hi, grb_hi)

        plsc.subcore_barrier()

        def wout(acc, o_lo, o_hi):
            @pl.when(c == 0)
            def _():
                pltpu.sync_copy(acc.at[pl.ds(s * NPT, NPT), :],
                                o_lo.at[pl.ds(s * NPT, NPT), :])

                @pl.when(s == 15)
                def _():
                    pltpu.sync_copy(acc.at[pl.ds(16 * NPT, 16), :],
                                    o_lo.at[pl.ds(16 * NPT, 16), :])

            @pl.when(c == 1)
            def _():
                pltpu.sync_copy(acc.at[pl.ds(s * NPT, NPT), :],
                                o_hi.at[pl.ds(s * NPT, NPT), :])

                @pl.when(s == 15)
                def _():
                    pltpu.sync_copy(acc.at[pl.ds(16 * NPT, 16), :],
                                    o_hi.at[pl.ds(16 * NPT, 16), :])

        wout(acc_u, ou_lo, ou_hi)
        wout(acc_i, oi_lo, oi_hi)

    quarter = jax.ShapeDtypeStruct((N, fh), jnp.float32)
    return pl.kernel(
        body,
        out_type=[quarter] * 4,
        mesh=_MESH,
        compiler_params=pltpu.CompilerParams(use_tc_tiling_on_sc=False),
        scratch_types=(
            [pltpu.VMEM((CH, IB), jnp.int32)] * 2 +
            [pltpu.VMEM((IB, fh), jnp.float32)] * NBUF2 +
            [pltpu.VMEM_SHARED((NA, fh), jnp.float32)] * 2 +
            [pltpu.SemaphoreType.DMA] * (2 * NBUF2)
        ),
    )


def _pad_edges(ei):
    r = jnp.arange(EPAD, dtype=jnp.int32)
    pad = jnp.stack([r % N, N + (r % (NA - N))])
    return jnp.concatenate([ei.astype(jnp.int32), pad], axis=1).reshape(
        2, ROWS, IB)


def kernel(x_user, x_item, W1_follows, W1_rates, W1_ratedby,
           W2_follows, W2_rates, W2_ratedby,
           ei_follows, ei_rates, ei_ratedby):
    ei_f = _pad_edges(ei_follows)
    ei_r = _pad_edges(ei_rates)
    ei_rb = _pad_edges(ei_ratedby)

    mm1a = _make_mm1a()
    mm1b = _make_mm1b()
    mm2a = _make_mm2a()
    mm2b = _make_mm2b()
    spmm2_h = _make_spmm_l1(2)
    spmm1_h = _make_spmm_l1(1)
    spmm_l2 = _make_spmm_l2()
    merge_relu = _make_merge_relu()

    hf = mm1a(x_user, x_item, W1_follows, W1_ratedby)
    hr = mm1b(x_user, W1_rates)
    hu1 = spmm2_h(*hf, ei_f, ei_rb)
    dep1 = (hu1[0][0, 0] * 0.0).astype(jnp.int32)
    hi1 = spmm1_h(*hr, ei_r + dep1)

    gf_lo, gf_hi, gr_lo, gr_hi = mm2a(*hu1, W2_follows, W2_rates)
    grb_lo, grb_hi = mm2b(*hi1, W2_ratedby)
    hu2_lo, hu2_hi, hi2_lo, hi2_hi = spmm_l2(
        gf_lo, gf_hi, gr_lo, gr_hi, grb_lo, grb_hi, ei_f, ei_r, ei_rb)

    return merge_relu(hu2_lo, hu2_hi, hi2_lo, hi2_hi)


# speedup vs baseline: 1.0714x; 1.0231x over previous
import jax
import jax.numpy as jnp
from jax import lax
from jax.experimental import pallas as pl
from jax.experimental.pallas import tpu as pltpu
from jax.experimental.pallas import tpu_sc as plsc

N = 10000
E = 160000
H = 256
OUT = 128
IB = 128
ROWS = 1280
EPAD = ROWS * IB - E
NA = N + 16
NPT = 624
NBUF = 2
NBUF2 = 4
CH = 40


def _mm1a_body(xu_ref, xi_ref, wf_ref, wrb_ref, *outs):
    fq = H // 4
    mf = jnp.dot(xu_ref[...], wf_ref[...], preferred_element_type=jnp.float32)
    mrb = jnp.dot(xi_ref[...], wrb_ref[...],
                  preferred_element_type=jnp.float32)
    for q in range(4):
        outs[q][...] = mf[:, q * fq:(q + 1) * fq]
        outs[4 + q][...] = mrb[:, q * fq:(q + 1) * fq]


def _make_mm1a():
    bm = 1000
    quarter = jax.ShapeDtypeStruct((N, H // 4), jnp.float32)
    return pl.pallas_call(
        _mm1a_body,
        grid=(N // bm,),
        in_specs=[
            pl.BlockSpec((bm, H), lambda i: (i, 0)),
            pl.BlockSpec((bm, H), lambda i: (i, 0)),
            pl.BlockSpec((H, H), lambda i: (0, 0)),
            pl.BlockSpec((H, H), lambda i: (0, 0)),
        ],
        out_specs=[pl.BlockSpec((bm, H // 4), lambda i: (i, 0))] * 8,
        out_shape=[quarter] * 8,
        compiler_params=pltpu.CompilerParams(
            dimension_semantics=("parallel",)),
    )


def _mm1b_body(xu_ref, wr_ref, *outs):
    fq = H // 4
    mr = jnp.dot(xu_ref[...], wr_ref[...], preferred_element_type=jnp.float32)
    for q in range(4):
        outs[q][...] = mr[:, q * fq:(q + 1) * fq]


def _make_mm1b():
    bm = 1000
    quarter = jax.ShapeDtypeStruct((N, H // 4), jnp.float32)
    return pl.pallas_call(
        _mm1b_body,
        grid=(N // bm,),
        in_specs=[
            pl.BlockSpec((bm, H), lambda i: (i, 0)),
            pl.BlockSpec((H, H), lambda i: (0, 0)),
        ],
        out_specs=[pl.BlockSpec((bm, H // 4), lambda i: (i, 0))] * 4,
        out_shape=[quarter] * 4,
        compiler_params=pltpu.CompilerParams(
            dimension_semantics=("parallel",)),
    )


def _mm2a_body(q0, q1, q2, q3, wf_ref, wr_ref, f_lo, f_hi, r_lo, r_hi):
    fh = OUT // 2
    xu = jnp.maximum(jnp.concatenate(
        [q0[...], q1[...], q2[...], q3[...]], axis=1), 0.0)
    mf = jnp.dot(xu, wf_ref[...], preferred_element_type=jnp.float32)
    mr = jnp.dot(xu, wr_ref[...], preferred_element_type=jnp.float32)
    f_lo[...] = mf[:, :fh]
    f_hi[...] = mf[:, fh:]
    r_lo[...] = mr[:, :fh]
    r_hi[...] = mr[:, fh:]


def _make_mm2a():
    bm = 1000
    quarter = jax.ShapeDtypeStruct((N, OUT // 2), jnp.float32)
    return pl.pallas_call(
        _mm2a_body,
        grid=(N // bm,),
        in_specs=[pl.BlockSpec((bm, H // 4), lambda i: (i, 0))] * 4 + [
            pl.BlockSpec((H, OUT), lambda i: (0, 0)),
            pl.BlockSpec((H, OUT), lambda i: (0, 0)),
        ],
        out_specs=[pl.BlockSpec((bm, OUT // 2), lambda i: (i, 0))] * 4,
        out_shape=[quarter] * 4,
        compiler_params=pltpu.CompilerParams(
            dimension_semantics=("parallel",)),
    )


def _mm2b_body(q0, q1, q2, q3, wrb_ref, rb_lo, rb_hi):
    fh = OUT // 2
    xi = jnp.maximum(jnp.concatenate(
        [q0[...], q1[...], q2[...], q3[...]], axis=1), 0.0)
    mrb = jnp.dot(xi, wrb_ref[...], preferred_element_type=jnp.float32)
    rb_lo[...] = mrb[:, :fh]
    rb_hi[...] = mrb[:, fh:]


def _make_mm2b():
    bm = 1000
    quarter = jax.ShapeDtypeStruct((N, OUT // 2), jnp.float32)
    return pl.pallas_call(
        _mm2b_body,
        grid=(N // bm,),
        in_specs=[pl.BlockSpec((bm, H // 4), lambda i: (i, 0))] * 4 + [
            pl.BlockSpec((H, OUT), lambda i: (0, 0)),
        ],
        out_specs=[pl.BlockSpec((bm, OUT // 2), lambda i: (i, 0))] * 2,
        out_shape=[quarter] * 2,
        compiler_params=pltpu.CompilerParams(
            dimension_semantics=("parallel",)),
    )


def _merge_relu_body(ul_ref, uh_ref, il_ref, ih_ref, ou_ref, oi_ref):
    ou_ref[...] = jnp.maximum(
        jnp.concatenate([ul_ref[...], uh_ref[...]], axis=1), 0.0)
    oi_ref[...] = jnp.maximum(
        jnp.concatenate([il_ref[...], ih_ref[...]], axis=1), 0.0)


def _make_merge_relu():
    bm = 1000
    shp = jax.ShapeDtypeStruct((N, OUT), jnp.float32)
    qspec = pl.BlockSpec((bm, OUT // 2), lambda i: (i, 0))
    return pl.pallas_call(
        _merge_relu_body,
        grid=(N // bm,),
        in_specs=[qspec] * 4,
        out_specs=[pl.BlockSpec((bm, OUT), lambda i: (i, 0))] * 2,
        out_shape=[shp, shp],
        compiler_params=pltpu.CompilerParams(
            dimension_semantics=("parallel",)),
    )


_MESH = plsc.VectorSubcoreMesh(core_axis_name="c", subcore_axis_name="s",
                               num_cores=2)


def _clear_accs(zbuf, accs, s, w):
    zero = jnp.zeros((16,), jnp.float32)

    def zrow(r, carry):
        for j in range(w // 16):
            zbuf[r, pl.ds(16 * j, 16)] = zero
        return carry

    lax.fori_loop(0, IB, zrow, 0)
    for acc in accs:
        for k in range(4):
            pltpu.sync_copy(zbuf.at[pl.ds(0, 128), :],
                            acc.at[pl.ds(s * NPT + k * 128, 128), :])
        pltpu.sync_copy(zbuf.at[pl.ds(0, 112), :],
                        acc.at[pl.ds(s * NPT + 512, 112), :])

        @pl.when(s == 15)
        def _():
            pltpu.sync_copy(zbuf.at[pl.ds(0, 16), :],
                            acc.at[pl.ds(16 * NPT, 16), :])


def _accumulate(h_ref, ei_ref, sidx, didx, rows, gsems, ssems, acc,
                base, nrows):
    nb = len(rows)
    ngrp = CH // nb

    def chunk(ci, carry):
        cbase = base + ci * CH
        pltpu.sync_copy(ei_ref.at[0, pl.ds(cbase, CH), :], sidx)
        pltpu.sync_copy(ei_ref.at[1, pl.ds(cbase, CH), :], didx)

        for b in range(nb):
            pltpu.async_copy(h_ref.at[sidx.at[b]], rows[b], gsems[b])

        def group(g, c2):
            t0 = g * nb
            for b in range(nb):
                pltpu.make_async_copy(h_ref.at[sidx.at[t0 + b]], rows[b],
                                      gsems[b]).wait()
                pltpu.async_copy(rows[b], acc.at[didx.at[t0 + b]], ssems[b],
                                 add=True)
            for b in range(nb):
                pltpu.make_async_copy(rows[b], acc.at[didx.at[t0 + b]],
                                      ssems[b]).wait()

                @pl.when(g + 1 < ngrp)
                def _():
                    pltpu.async_copy(h_ref.at[sidx.at[t0 + nb + b]],
                                     rows[b], gsems[b])
            return c2

        lax.fori_loop(0, ngrp, group, 0)
        return carry

    lax.fori_loop(0, nrows // CH, chunk, 0)


def _accumulate2(ha, hb, ei_ref, sidx, didx, rows, gsems, ssems,
                 acc_a, acc_b, base, nrows):
    hsel = [ha, hb] * (NBUF2 // 2)
    asel = [acc_a, acc_b] * (NBUF2 // 2)
    ngrp = CH // (NBUF2 // 2)

    def chunk(ci, carry):
        cbase = base + ci * CH
        pltpu.sync_copy(ei_ref.at[0, pl.ds(cbase, CH), :], sidx)
        pltpu.sync_copy(ei_ref.at[1, pl.ds(cbase, CH), :], didx)

        for b in range(NBUF2):
            pltpu.async_copy(hsel[b].at[sidx.at[b // 2]], rows[b], gsems[b])

        def group(g, c2):
            t0 = g * (NBUF2 // 2)
            for b in range(NBUF2):
                pltpu.make_async_copy(hsel[b].at[sidx.at[t0 + b // 2]],
                                      rows[b], gsems[b]).wait()
                pltpu.async_copy(rows[b], asel[b].at[didx.at[t0 + b // 2]],
                                 ssems[b], add=True)
            for b in range(NBUF2):
                pltpu.make_async_copy(rows[b], asel[b].at[didx.at[t0 + b // 2]],
                                      ssems[b]).wait()

                @pl.when(g + 1 < ngrp)
                def _():
                    pltpu.async_copy(
                        hsel[b].at[sidx.at[t0 + NBUF2 // 2 + b // 2]],
                        rows[b], gsems[b])
            return c2

        lax.fori_loop(0, ngrp, group, 0)
        return carry

    lax.fori_loop(0, nrows // CH, chunk, 0)


def _make_spmm_l1(n_rel):
    fq = H // 4
    rpt = ROWS // 16

    def body(*refs):
        nq = 4 * n_rel
        h_refs = refs[:nq]
        ei_refs = refs[nq:nq + n_rel]
        out_refs = refs[nq + n_rel:nq + n_rel + 4]
        sidx = refs[nq + n_rel + 4]
        didx = refs[nq + n_rel + 5]
        rest = refs[nq + n_rel + 6:]
        rows = rest[:NBUF2]
        acc_a = rest[NBUF2]
        acc_b = rest[NBUF2 + 1]
        gsems = rest[NBUF2 + 2:NBUF2 + 2 + NBUF2]
        ssems = rest[NBUF2 + 2 + NBUF2:]

        c = lax.axis_index("c")
        s = lax.axis_index("s")

        _clear_accs(rows[0], [acc_a, acc_b], s, fq)
        plsc.subcore_barrier()

        def run(qoff):
            for r in range(n_rel):
                _accumulate2(h_refs[4 * r + qoff], h_refs[4 * r + qoff + 1],
                             ei_refs[r], sidx, didx, rows, gsems, ssems,
                             acc_a, acc_b, s * rpt, rpt)

        @pl.when(c == 0)
        def _():
            run(0)

        @pl.when(c == 1)
        def _():
            run(2)

        plsc.subcore_barrier()

        def wout(acc, o0, o1):
            @pl.when(c == 0)
            def _():
                pltpu.sync_copy(acc.at[pl.ds(s * NPT, NPT), :],
                                o0.at[pl.ds(s * NPT, NPT), :])

                @pl.when(s == 15)
                def _():
                    pltpu.sync_copy(acc.at[pl.ds(16 * NPT, 16), :],
                                    o0.at[pl.ds(16 * NPT, 16), :])

            @pl.when(c == 1)
            def _():
                pltpu.sync_copy(acc.at[pl.ds(s * NPT, NPT), :],
                                o1.at[pl.ds(s * NPT, NPT), :])

                @pl.when(s == 15)
                def _():
                    pltpu.sync_copy(acc.at[pl.ds(16 * NPT, 16), :],
                                    o1.at[pl.ds(16 * NPT, 16), :])

        wout(acc_a, out_refs[0], out_refs[2])
        wout(acc_b, out_refs[1], out_refs[3])

    quarter = jax.ShapeDtypeStruct((N, fq), jnp.float32)
    return pl.kernel(
        body,
        out_type=[quarter] * 4,
        mesh=_MESH,
        compiler_params=pltpu.CompilerParams(use_tc_tiling_on_sc=False),
        scratch_types=(
            [pltpu.VMEM((CH, IB), jnp.int32)] * 2 +
            [pltpu.VMEM((IB, fq), jnp.float32)] * NBUF2 +
            [pltpu.VMEM_SHARED((NA, fq), jnp.float32)] * 2 +
            [pltpu.SemaphoreType.DMA] * (2 * NBUF2)
        ),
    )


def _make_spmm_l2():
    fh = OUT // 2
    rpt = ROWS // 16

    def body(gf_lo, gf_hi, gr_lo, gr_hi, grb_lo, grb_hi,
             ei_f, ei_r, ei_rb, ou_lo, ou_hi, oi_lo, oi_hi,
             sidx, didx, *rest):
        rows = rest[:NBUF2]
        acc_u = rest[NBUF2]
        acc_i = rest[NBUF2 + 1]
        gsems = rest[NBUF2 + 2:NBUF2 + 2 + NBUF2]
        ssems = rest[NBUF2 + 2 + NBUF2:]

        c = lax.axis_index("c")
        s = lax.axis_index("s")

        _clear_accs(rows[0], [acc_u, acc_i], s, fh)
        plsc.subcore_barrier()

        def run(gf, gr, grb):
            _accumulate(gf, ei_f, sidx, didx, rows, gsems, ssems, acc_u,
                        s * rpt, rpt)
            _accumulate(grb, ei_rb, sidx, didx, rows, gsems, ssems, acc_u,
                        s * rpt, rpt)
            _accumulate(gr, ei_r, sidx, didx, rows, gsems, ssems, acc_i,
                        s * rpt, rpt)

        @pl.when(c == 0)
        def _():
            run(gf_lo, gr_lo, grb_lo)

        @pl.when(c == 1)
        def _():
            run(gf_hi, gr_hi, grb_hi)

        plsc.subcore_barrier()

        def wout(acc, o_lo, o_hi):
            @pl.when(c == 0)
            def _():
                pltpu.sync_copy(acc.at[pl.ds(s * NPT, NPT), :],
                                o_lo.at[pl.ds(s * NPT, NPT), :])

                @pl.when(s == 15)
                def _():
                    pltpu.sync_copy(acc.at[pl.ds(16 * NPT, 16), :],
                                    o_lo.at[pl.ds(16 * NPT, 16), :])

            @pl.when(c == 1)
            def _():
                pltpu.sync_copy(acc.at[pl.ds(s * NPT, NPT), :],
                                o_hi.at[pl.ds(s * NPT, NPT), :])

                @pl.when(s == 15)
                def _():
                    pltpu.sync_copy(acc.at[pl.ds(16 * NPT, 16), :],
                                    o_hi.at[pl.ds(16 * NPT, 16), :])

        wout(acc_u, ou_lo, ou_hi)
        wout(acc_i, oi_lo, oi_hi)

    quarter = jax.ShapeDtypeStruct((N, fh), jnp.float32)
    return pl.kernel(
        body,
        out_type=[quarter] * 4,
        mesh=_MESH,
        compiler_params=pltpu.CompilerParams(use_tc_tiling_on_sc=False),
        scratch_types=(
            [pltpu.VMEM((CH, IB), jnp.int32)] * 2 +
            [pltpu.VMEM((IB, fh), jnp.float32)] * NBUF2 +
            [pltpu.VMEM_SHARED((NA, fh), jnp.float32)] * 2 +
            [pltpu.SemaphoreType.DMA] * (2 * NBUF2)
        ),
    )


def _pad_edges(ei):
    r = jnp.arange(EPAD, dtype=jnp.int32)
    pad = jnp.stack([r % N, N + (r % (NA - N))])
    return jnp.concatenate([ei.astype(jnp.int32), pad], axis=1).reshape(
        2, ROWS, IB)


def kernel(x_user, x_item, W1_follows, W1_rates, W1_ratedby,
           W2_follows, W2_rates, W2_ratedby,
           ei_follows, ei_rates, ei_ratedby):
    ei_f = _pad_edges(ei_follows)
    ei_r = _pad_edges(ei_rates)
    ei_rb = _pad_edges(ei_ratedby)

    mm1a = _make_mm1a()
    mm1b = _make_mm1b()
    mm2a = _make_mm2a()
    mm2b = _make_mm2b()
    spmm2_h = _make_spmm_l1(2)
    spmm1_h = _make_spmm_l1(1)
    spmm_l2 = _make_spmm_l2()
    merge_relu = _make_merge_relu()

    hf = mm1a(x_user, x_item, W1_follows, W1_ratedby)
    hr = mm1b(x_user, W1_rates)
    hu1 = spmm2_h(*hf, ei_f, ei_rb)
    dep1 = (hu1[0][0, 0] * 0.0).astype(jnp.int32)
    hi1 = spmm1_h(*hr, ei_r + dep1)

    gf_lo, gf_hi, gr_lo, gr_hi = mm2a(*hu1, W2_follows, W2_rates)
    grb_lo, grb_hi = mm2b(*hi1, W2_ratedby)
    hu2_lo, hu2_hi, hi2_lo, hi2_hi = spmm_l2(
        gf_lo, gf_hi, gr_lo, gr_hi, grb_lo, grb_hi, ei_f, ei_r, ei_rb)

    return merge_relu(hu2_lo, hu2_hi, hi2_lo, hi2_hi)
